# Initial kernel scaffold; baseline (speedup 1.0000x reference)
import jax
import jax.numpy as jnp
from jax.experimental import pallas as pl


def kernel(x, edge_index, W1, b1, W2, b2, temp):
    raise NotImplementedError("write your pallas kernel here")


# speedup vs baseline: 8.0655x; 8.0655x over previous
import functools

import jax
import jax.numpy as jnp
from jax import lax
from jax.experimental import pallas as pl
from jax.experimental.pallas import tpu as pltpu
from jax.experimental.pallas import tpu_sc as plsc

N = 10000
E = 320000
D = 128
H = 128
C = 64
K = 10

N_PAD = 10240
TRASH = N
NW = 32
EPC = 128
NCHUNK = 80
EPT = EPC * NCHUNK
E_PAD = EPT * NW
ROWS_PER_TILE = N_PAD // 16
DEG_W = 16

_mesh = plsc.VectorSubcoreMesh(core_axis_name="c", subcore_axis_name="s")


@functools.partial(
    pl.kernel,
    out_type=jax.ShapeDtypeStruct((2, N_PAD, DEG_W), jnp.float32),
    mesh=_mesh,
    compiler_params=pltpu.CompilerParams(use_tc_tiling_on_sc=False),
    scratch_types=[
        pltpu.VMEM((NCHUNK, EPC), jnp.int32),
        pltpu.VMEM((EPC, DEG_W), jnp.float32),
        pltpu.VMEM((ROWS_PER_TILE, DEG_W), jnp.float32),
        pltpu.VMEM_SHARED((N_PAD, DEG_W), jnp.float32),
    ],
)
def _deg_kernel(col_hbm, pd_hbm, col_v, ones_v, zero_v, acc):
    cid = lax.axis_index("c")
    sid = lax.axis_index("s")
    wid = sid * 2 + cid

    pltpu.sync_copy(col_hbm.at[wid], col_v)

    one16 = jnp.full((16,), 1.0, dtype=jnp.float32)
    zero16 = jnp.zeros((16,), dtype=jnp.float32)

    def fill_ones(i, carry):
        ones_v[i, :] = one16
        return carry

    lax.fori_loop(0, EPC, fill_ones, 0)

    def fill_zero(i, carry):
        zero_v[i, :] = zero16
        return carry

    lax.fori_loop(0, ROWS_PER_TILE, fill_zero, 0)

    pltpu.sync_copy(zero_v, acc.at[pl.ds(sid * ROWS_PER_TILE, ROWS_PER_TILE)])
    plsc.subcore_barrier()

    def step(ci, carry):
        pltpu.sync_copy(ones_v, acc.at[col_v.at[ci]], add=True)
        return carry

    lax.fori_loop(0, NCHUNK, step, 0)

    plsc.subcore_barrier()
    pltpu.sync_copy(
        acc.at[pl.ds(sid * ROWS_PER_TILE, ROWS_PER_TILE)],
        pd_hbm.at[cid, pl.ds(sid * ROWS_PER_TILE, ROWS_PER_TILE)],
    )


@functools.partial(
    pl.kernel,
    out_type=jax.ShapeDtypeStruct((2, N_PAD, C), jnp.float32),
    mesh=_mesh,
    compiler_params=pltpu.CompilerParams(use_tc_tiling_on_sc=False),
    scratch_types=[
        pltpu.VMEM((NCHUNK, EPC), jnp.int32),
        pltpu.VMEM((NCHUNK, EPC), jnp.int32),
        pltpu.VMEM((EPC, C), jnp.float32),
        pltpu.VMEM((EPC, C), jnp.float32),
        pltpu.VMEM((ROWS_PER_TILE, C), jnp.float32),
        pltpu.VMEM_SHARED((N_PAD, C), jnp.float32),
        pltpu.SemaphoreType.DMA,
        pltpu.SemaphoreType.DMA,
    ],
)
def _hop_kernel(g_hbm, row_hbm, col_hbm, p_hbm,
                row_v, col_v, buf0, buf1, zero_v, acc, sem0, sem1):
    cid = lax.axis_index("c")
    sid = lax.axis_index("s")
    wid = sid * 2 + cid

    pltpu.sync_copy(row_hbm.at[wid], row_v)
    pltpu.sync_copy(col_hbm.at[wid], col_v)

    zero16 = jnp.zeros((16,), dtype=jnp.float32)

    def fill_zero(i, carry):
        for j in range(C // 16):
            zero_v[i, pl.ds(j * 16, 16)] = zero16
        return carry

    lax.fori_loop(0, ROWS_PER_TILE, fill_zero, 0)

    pltpu.sync_copy(zero_v, acc.at[pl.ds(sid * ROWS_PER_TILE, ROWS_PER_TILE)])
    plsc.subcore_barrier()

    def pair(i, carry):
        c0 = i * 2
        c1 = c0 + 1
        pltpu.async_copy(g_hbm.at[row_v.at[c0]], buf0, sem0)
        pltpu.async_copy(g_hbm.at[row_v.at[c1]], buf1, sem1)
        pltpu.make_async_copy(g_hbm.at[row_v.at[c0]], buf0, sem0).wait()
        pltpu.sync_copy(buf0, acc.at[col_v.at[c0]], add=True)
        pltpu.make_async_copy(g_hbm.at[row_v.at[c1]], buf1, sem1).wait()
        pltpu.sync_copy(buf1, acc.at[col_v.at[c1]], add=True)
        return carry

    lax.fori_loop(0, NCHUNK // 2, pair, 0)

    plsc.subcore_barrier()
    pltpu.sync_copy(
        acc.at[pl.ds(sid * ROWS_PER_TILE, ROWS_PER_TILE)],
        p_hbm.at[cid, pl.ds(sid * ROWS_PER_TILE, ROWS_PER_TILE)],
    )


_BLK = 512
_NBLK = N_PAD // _BLK


def _linear_body(x_ref, w1_ref, b1_ref, w2_ref, b2_ref, pd_ref, t0_ref,
                 g_ref, hid_ref, deg_ref):
    x = x_ref[...]
    h1 = lax.dot_general(x, w1_ref[...], (((1,), (1,)), ((), ())),
                         preferred_element_type=jnp.float32)
    h1 = jnp.maximum(h1 + b1_ref[...], 0.0)
    h = lax.dot_general(h1, w2_ref[...], (((1,), (1,)), ((), ())),
                        preferred_element_type=jnp.float32)
    h = h + b2_ref[...]
    deg = 1.0 + pd_ref[0, :, 0:1] + pd_ref[1, :, 0:1]
    dinv = lax.rsqrt(deg)
    hid_ref[...] = t0_ref[0, 0] * h
    g_ref[...] = dinv * h
    deg_ref[...] = deg


_linear_call = pl.pallas_call(
    _linear_body,
    grid=(_NBLK,),
    in_specs=[
        pl.BlockSpec((_BLK, D), lambda i: (i, 0)),
        pl.BlockSpec((H, D), lambda i: (0, 0)),
        pl.BlockSpec((1, H), lambda i: (0, 0)),
        pl.BlockSpec((C, H), lambda i: (0, 0)),
        pl.BlockSpec((1, C), lambda i: (0, 0)),
        pl.BlockSpec((2, _BLK, DEG_W), lambda i: (0, i, 0)),
        pl.BlockSpec((1, 1), lambda i: (0, 0)),
    ],
    out_specs=[
        pl.BlockSpec((_BLK, C), lambda i: (i, 0)),
        pl.BlockSpec((_BLK, C), lambda i: (i, 0)),
        pl.BlockSpec((_BLK, 1), lambda i: (i, 0)),
    ],
    out_shape=[
        jax.ShapeDtypeStruct((N_PAD, C), jnp.float32),
        jax.ShapeDtypeStruct((N_PAD, C), jnp.float32),
        jax.ShapeDtypeStruct((N_PAD, 1), jnp.float32),
    ],
)


def _combine_body(p_ref, g_ref, deg_ref, hid_ref, t_ref, gout_ref, hidout_ref):
    s = p_ref[0] + p_ref[1] + g_ref[...]
    dinv = lax.rsqrt(deg_ref[...])
    hidout_ref[...] = hid_ref[...] + t_ref[0, 0] * (dinv * s)
    gout_ref[...] = (dinv * dinv) * s


def _combine_final_body(p_ref, g_ref, deg_ref, hid_ref, t_ref, out_ref):
    s = p_ref[0] + p_ref[1] + g_ref[...]
    dinv = lax.rsqrt(deg_ref[...])
    hid = hid_ref[...] + t_ref[0, 0] * (dinv * s)
    m = jnp.max(hid, axis=1, keepdims=True)
    ex = jnp.exp(hid - m)
    lse = m + jnp.log(jnp.sum(ex, axis=1, keepdims=True))
    out_ref[...] = hid - lse


_combine_in_specs = [
    pl.BlockSpec((2, _BLK, C), lambda i: (0, i, 0)),
    pl.BlockSpec((_BLK, C), lambda i: (i, 0)),
    pl.BlockSpec((_BLK, 1), lambda i: (i, 0)),
    pl.BlockSpec((_BLK, C), lambda i: (i, 0)),
    pl.BlockSpec((1, 1), lambda i: (0, 0)),
]

_combine_call = pl.pallas_call(
    _combine_body,
    grid=(_NBLK,),
    in_specs=_combine_in_specs,
    out_specs=[
        pl.BlockSpec((_BLK, C), lambda i: (i, 0)),
        pl.BlockSpec((_BLK, C), lambda i: (i, 0)),
    ],
    out_shape=[
        jax.ShapeDtypeStruct((N_PAD, C), jnp.float32),
        jax.ShapeDtypeStruct((N_PAD, C), jnp.float32),
    ],
)

_combine_final_call = pl.pallas_call(
    _combine_final_body,
    grid=(_NBLK,),
    in_specs=_combine_in_specs,
    out_specs=pl.BlockSpec((_BLK, C), lambda i: (i, 0)),
    out_shape=jax.ShapeDtypeStruct((N_PAD, C), jnp.float32),
)


def kernel(x, edge_index, W1, b1, W2, b2, temp):
    row = edge_index[0]
    col = edge_index[1]
    col_eff = jnp.where(row == col, TRASH, col).astype(jnp.int32)

    pad = E_PAD - E
    row_p = jnp.concatenate(
        [row.astype(jnp.int32), jnp.zeros((pad,), jnp.int32)]
    ).reshape(NW, NCHUNK, EPC)
    col_p = jnp.concatenate(
        [col_eff, jnp.full((pad,), TRASH, jnp.int32)]
    ).reshape(NW, NCHUNK, EPC)

    x_p = jnp.pad(x, ((0, N_PAD - N), (0, 0)))

    pd = _deg_kernel(col_p)
    g, hid, deg = _linear_call(
        x_p, W1, b1.reshape(1, H), W2, b2.reshape(1, C), pd,
        temp[0].reshape(1, 1),
    )

    for k in range(K):
        p = _hop_kernel(g, row_p, col_p)
        t_k = temp[k + 1].reshape(1, 1)
        if k < K - 1:
            g, hid = _combine_call(p, g, deg, hid, t_k)
        else:
            out = _combine_final_call(p, g, deg, hid, t_k)

    return out[:N]


# speedup vs baseline: 8.8959x; 1.1030x over previous
import functools

import jax
import jax.numpy as jnp
from jax import lax
from jax.experimental import pallas as pl
from jax.experimental.pallas import tpu as pltpu
from jax.experimental.pallas import tpu_sc as plsc

N = 10000
E = 320000
D = 128
H = 128
C = 64
K = 10

N_PAD = 10240
TRASH = N
NW = 32
EPC = 128
NCHUNK = 80
EPT = EPC * NCHUNK
E_PAD = EPT * NW
ROWS_PER_TILE = N_PAD // 16
DEG_W = 16

_mesh = plsc.VectorSubcoreMesh(core_axis_name="c", subcore_axis_name="s")


@functools.partial(
    pl.kernel,
    out_type=jax.ShapeDtypeStruct((2, N_PAD, DEG_W), jnp.float32),
    mesh=_mesh,
    compiler_params=pltpu.CompilerParams(use_tc_tiling_on_sc=False),
    scratch_types=[
        pltpu.VMEM((NCHUNK, EPC), jnp.int32),
        pltpu.VMEM((EPC, DEG_W), jnp.float32),
        pltpu.VMEM((ROWS_PER_TILE, DEG_W), jnp.float32),
        pltpu.VMEM_SHARED((N_PAD, DEG_W), jnp.float32),
    ],
)
def _deg_kernel(col_hbm, pd_hbm, col_v, ones_v, zero_v, acc):
    cid = lax.axis_index("c")
    sid = lax.axis_index("s")
    wid = sid * 2 + cid

    pltpu.sync_copy(col_hbm.at[wid], col_v)

    one16 = jnp.full((16,), 1.0, dtype=jnp.float32)
    zero16 = jnp.zeros((16,), dtype=jnp.float32)

    def fill_ones(i, carry):
        ones_v[i, :] = one16
        return carry

    lax.fori_loop(0, EPC, fill_ones, 0)

    def fill_zero(i, carry):
        zero_v[i, :] = zero16
        return carry

    lax.fori_loop(0, ROWS_PER_TILE, fill_zero, 0)

    pltpu.sync_copy(zero_v, acc.at[pl.ds(sid * ROWS_PER_TILE, ROWS_PER_TILE)])
    plsc.subcore_barrier()

    def step(ci, carry):
        pltpu.sync_copy(ones_v, acc.at[col_v.at[ci]], add=True)
        return carry

    lax.fori_loop(0, NCHUNK, step, 0)

    plsc.subcore_barrier()
    pltpu.sync_copy(
        acc.at[pl.ds(sid * ROWS_PER_TILE, ROWS_PER_TILE)],
        pd_hbm.at[cid, pl.ds(sid * ROWS_PER_TILE, ROWS_PER_TILE)],
    )


NBUF = 8
NSTEP = NCHUNK // NBUF
ZROWS = 64


@functools.partial(
    pl.kernel,
    out_type=jax.ShapeDtypeStruct((2, N_PAD, C), jnp.float32),
    mesh=_mesh,
    compiler_params=pltpu.CompilerParams(use_tc_tiling_on_sc=False),
    scratch_types=[
        pltpu.VMEM((NCHUNK, EPC), jnp.int32),
        pltpu.VMEM((NCHUNK, EPC), jnp.int32),
        [pltpu.VMEM((EPC, C), jnp.float32)] * NBUF,
        pltpu.VMEM((ZROWS, C), jnp.float32),
        pltpu.VMEM_SHARED((N_PAD, C), jnp.float32),
        [pltpu.SemaphoreType.DMA] * NBUF,
        [pltpu.SemaphoreType.DMA] * NBUF,
    ],
)
def _hop_kernel(g_hbm, row_hbm, col_hbm, p_hbm,
                row_v, col_v, bufs, zero_v, acc, gsem, ssem):
    cid = lax.axis_index("c")
    sid = lax.axis_index("s")
    wid = sid * 2 + cid

    pltpu.sync_copy(row_hbm.at[wid], row_v)
    pltpu.sync_copy(col_hbm.at[wid], col_v)

    zero16 = jnp.zeros((16,), dtype=jnp.float32)

    def fill_zero(i, carry):
        for j in range(C // 16):
            zero_v[i, pl.ds(j * 16, 16)] = zero16
        return carry

    lax.fori_loop(0, ZROWS, fill_zero, 0)

    for z in range(ROWS_PER_TILE // ZROWS):
        pltpu.sync_copy(
            zero_v,
            acc.at[pl.ds(sid * ROWS_PER_TILE + z * ZROWS, ZROWS)],
        )
    plsc.subcore_barrier()

    for b in range(NBUF):
        pltpu.async_copy(g_hbm.at[row_v.at[b]], bufs[b], gsem[b])

    def step(s, carry):
        for b in range(NBUF):
            c = s * NBUF + b
            pltpu.make_async_copy(g_hbm.at[row_v.at[c]], bufs[b], gsem[b]).wait()
            pltpu.async_copy(bufs[b], acc.at[col_v.at[c]], ssem[b], add=True)
        for b in range(NBUF):
            c = s * NBUF + b
            pltpu.make_async_copy(bufs[b], acc.at[col_v.at[c]], ssem[b]).wait()
            pltpu.async_copy(g_hbm.at[row_v.at[(s + 1) * NBUF + b]],
                             bufs[b], gsem[b])
        return carry

    lax.fori_loop(0, NSTEP - 1, step, 0)

    last = (NSTEP - 1) * NBUF
    for b in range(NBUF):
        pltpu.make_async_copy(g_hbm.at[row_v.at[last + b]], bufs[b],
                              gsem[b]).wait()
        pltpu.async_copy(bufs[b], acc.at[col_v.at[last + b]], ssem[b],
                         add=True)
    for b in range(NBUF):
        pltpu.make_async_copy(bufs[b], acc.at[col_v.at[last + b]],
                              ssem[b]).wait()

    plsc.subcore_barrier()
    pltpu.sync_copy(
        acc.at[pl.ds(sid * ROWS_PER_TILE, ROWS_PER_TILE)],
        p_hbm.at[cid, pl.ds(sid * ROWS_PER_TILE, ROWS_PER_TILE)],
    )


_BLK = 512
_NBLK = N_PAD // _BLK


def _linear_body(x_ref, w1_ref, b1_ref, w2_ref, b2_ref, pd_ref, t0_ref,
                 g_ref, hid_ref, deg_ref):
    x = x_ref[...]
    h1 = lax.dot_general(x, w1_ref[...], (((1,), (1,)), ((), ())),
                         preferred_element_type=jnp.float32)
    h1 = jnp.maximum(h1 + b1_ref[...], 0.0)
    h = lax.dot_general(h1, w2_ref[...], (((1,), (1,)), ((), ())),
                        preferred_element_type=jnp.float32)
    h = h + b2_ref[...]
    deg = 1.0 + pd_ref[0, :, 0:1] + pd_ref[1, :, 0:1]
    dinv = lax.rsqrt(deg)
    hid_ref[...] = t0_ref[0, 0] * h
    g_ref[...] = dinv * h
    deg_ref[...] = deg


_linear_call = pl.pallas_call(
    _linear_body,
    grid=(_NBLK,),
    in_specs=[
        pl.BlockSpec((_BLK, D), lambda i: (i, 0)),
        pl.BlockSpec((H, D), lambda i: (0, 0)),
        pl.BlockSpec((1, H), lambda i: (0, 0)),
        pl.BlockSpec((C, H), lambda i: (0, 0)),
        pl.BlockSpec((1, C), lambda i: (0, 0)),
        pl.BlockSpec((2, _BLK, DEG_W), lambda i: (0, i, 0)),
        pl.BlockSpec((1, 1), lambda i: (0, 0)),
    ],
    out_specs=[
        pl.BlockSpec((_BLK, C), lambda i: (i, 0)),
        pl.BlockSpec((_BLK, C), lambda i: (i, 0)),
        pl.BlockSpec((_BLK, 1), lambda i: (i, 0)),
    ],
    out_shape=[
        jax.ShapeDtypeStruct((N_PAD, C), jnp.float32),
        jax.ShapeDtypeStruct((N_PAD, C), jnp.float32),
        jax.ShapeDtypeStruct((N_PAD, 1), jnp.float32),
    ],
)


def _combine_body(p_ref, g_ref, deg_ref, hid_ref, t_ref, gout_ref, hidout_ref):
    s = p_ref[0] + p_ref[1] + g_ref[...]
    dinv = lax.rsqrt(deg_ref[...])
    hidout_ref[...] = hid_ref[...] + t_ref[0, 0] * (dinv * s)
    gout_ref[...] = (dinv * dinv) * s


def _combine_final_body(p_ref, g_ref, deg_ref, hid_ref, t_ref, out_ref):
    s = p_ref[0] + p_ref[1] + g_ref[...]
    dinv = lax.rsqrt(deg_ref[...])
    hid = hid_ref[...] + t_ref[0, 0] * (dinv * s)
    m = jnp.max(hid, axis=1, keepdims=True)
    ex = jnp.exp(hid - m)
    lse = m + jnp.log(jnp.sum(ex, axis=1, keepdims=True))
    out_ref[...] = hid - lse


_combine_in_specs = [
    pl.BlockSpec((2, _BLK, C), lambda i: (0, i, 0)),
    pl.BlockSpec((_BLK, C), lambda i: (i, 0)),
    pl.BlockSpec((_BLK, 1), lambda i: (i, 0)),
    pl.BlockSpec((_BLK, C), lambda i: (i, 0)),
    pl.BlockSpec((1, 1), lambda i: (0, 0)),
]

_combine_call = pl.pallas_call(
    _combine_body,
    grid=(_NBLK,),
    in_specs=_combine_in_specs,
    out_specs=[
        pl.BlockSpec((_BLK, C), lambda i: (i, 0)),
        pl.BlockSpec((_BLK, C), lambda i: (i, 0)),
    ],
    out_shape=[
        jax.ShapeDtypeStruct((N_PAD, C), jnp.float32),
        jax.ShapeDtypeStruct((N_PAD, C), jnp.float32),
    ],
)

_combine_final_call = pl.pallas_call(
    _combine_final_body,
    grid=(_NBLK,),
    in_specs=_combine_in_specs,
    out_specs=pl.BlockSpec((_BLK, C), lambda i: (i, 0)),
    out_shape=jax.ShapeDtypeStruct((N_PAD, C), jnp.float32),
)


def kernel(x, edge_index, W1, b1, W2, b2, temp):
    row = edge_index[0]
    col = edge_index[1]
    col_eff = jnp.where(row == col, TRASH, col).astype(jnp.int32)

    pad = E_PAD - E
    row_p = jnp.concatenate(
        [row.astype(jnp.int32), jnp.zeros((pad,), jnp.int32)]
    ).reshape(NW, NCHUNK, EPC)
    col_p = jnp.concatenate(
        [col_eff, jnp.full((pad,), TRASH, jnp.int32)]
    ).reshape(NW, NCHUNK, EPC)

    x_p = jnp.pad(x, ((0, N_PAD - N), (0, 0)))

    pd = _deg_kernel(col_p)
    g, hid, deg = _linear_call(
        x_p, W1, b1.reshape(1, H), W2, b2.reshape(1, C), pd,
        temp[0].reshape(1, 1),
    )

    for k in range(K):
        p = _hop_kernel(g, row_p, col_p)
        t_k = temp[k + 1].reshape(1, 1)
        if k < K - 1:
            g, hid = _combine_call(p, g, deg, hid, t_k)
        else:
            out = _combine_final_call(p, g, deg, hid, t_k)

    return out[:N]
